# Initial kernel scaffold; baseline (speedup 1.0000x reference)
import jax
import jax.numpy as jnp
from jax.experimental import pallas as pl


def kernel(input_ids, freeze_buffer, training_part):
    raise NotImplementedError("write your pallas kernel here")


# speedup vs baseline: 1.0603x; 1.0603x over previous
import functools

import jax
import jax.numpy as jnp
from jax import lax
from jax.experimental import pallas as pl
from jax.experimental.pallas import tpu as pltpu
from jax.experimental.pallas import tpu_sc as plsc

_NUM_CORES = 2
_NUM_SUBCORES = 16
_NUM_WORKERS = _NUM_CORES * _NUM_SUBCORES
_CHUNK = 128


@functools.lru_cache(maxsize=None)
def _make_sc_gather(n_idx, d_freeze, d_train):
    d_out = d_freeze + d_train
    bpw = n_idx // _NUM_WORKERS
    n_chunks = bpw // _CHUNK
    assert n_idx % _NUM_WORKERS == 0 and bpw % _CHUNK == 0

    mesh = plsc.VectorSubcoreMesh(
        core_axis_name="c", subcore_axis_name="s",
        num_cores=_NUM_CORES, num_subcores=_NUM_SUBCORES)

    @functools.partial(
        pl.kernel,
        out_type=jax.ShapeDtypeStruct((n_idx, d_out), jnp.float32),
        mesh=mesh,
        scratch_types=[
            pltpu.VMEM((bpw,), jnp.int32),
            pltpu.VMEM((_CHUNK, d_freeze), jnp.float32),
            pltpu.VMEM((_CHUNK, d_train), jnp.float32),
            pltpu.SemaphoreType.DMA,
            pltpu.SemaphoreType.DMA,
        ],
        compiler_params=pltpu.CompilerParams(use_tc_tiling_on_sc=False),
    )
    def sc_gather(freeze_hbm, train_hbm, idx_hbm, out_hbm,
                  idx_v, rows_f, rows_t, sem_f, sem_t):
        wid = lax.axis_index("s") * _NUM_CORES + lax.axis_index("c")
        base = wid * bpw
        pltpu.sync_copy(idx_hbm.at[pl.ds(base, bpw)], idx_v)

        def chunk_body(j, carry):
            off = j * _CHUNK
            idx_c = idx_v.at[pl.ds(off, _CHUNK)]
            cf = pltpu.async_copy(freeze_hbm.at[idx_c], rows_f, sem_f)
            ct = pltpu.async_copy(train_hbm.at[idx_c], rows_t, sem_t)
            cf.wait()
            ct.wait()
            row0 = base + off
            pltpu.sync_copy(rows_f,
                            out_hbm.at[pl.ds(row0, _CHUNK), pl.ds(0, d_freeze)])
            pltpu.sync_copy(rows_t,
                            out_hbm.at[pl.ds(row0, _CHUNK), pl.ds(d_freeze, d_train)])
            return carry

        lax.fori_loop(0, n_chunks, chunk_body, 0)

    return sc_gather


def kernel(input_ids, freeze_buffer, training_part):
    b, s = input_ids.shape
    n_idx = b * s
    d_freeze = freeze_buffer.shape[1]
    d_train = training_part.shape[1]
    idx = jnp.reshape(input_ids, (n_idx,)).astype(jnp.int32)
    sc_gather = _make_sc_gather(n_idx, d_freeze, d_train)
    out = sc_gather(freeze_buffer, training_part, idx)
    return jnp.reshape(out, (b, s, d_freeze + d_train))


# speedup vs baseline: 1.0979x; 1.0354x over previous
import functools

import jax
import jax.numpy as jnp
from jax import lax
from jax.experimental import pallas as pl
from jax.experimental.pallas import tpu as pltpu
from jax.experimental.pallas import tpu_sc as plsc

_NUM_CORES = 2
_NUM_SUBCORES = 16
_NUM_WORKERS = _NUM_CORES * _NUM_SUBCORES
_CHUNK = 104


@functools.lru_cache(maxsize=None)
def _make_sc_gather(n_idx, d_freeze, d_train):
    d_out = d_freeze + d_train
    bpw = n_idx // _NUM_WORKERS
    n_chunks = bpw // _CHUNK
    assert n_idx % _NUM_WORKERS == 0 and bpw % _CHUNK == 0

    mesh = plsc.VectorSubcoreMesh(
        core_axis_name="c", subcore_axis_name="s",
        num_cores=_NUM_CORES, num_subcores=_NUM_SUBCORES)

    nbuf = 4
    assert n_chunks % nbuf == 0

    @functools.partial(
        pl.kernel,
        out_type=jax.ShapeDtypeStruct((n_idx, d_out), jnp.float32),
        mesh=mesh,
        scratch_types=[
            pltpu.VMEM((bpw,), jnp.int32),
            pltpu.VMEM((nbuf, _CHUNK, d_freeze), jnp.float32),
            pltpu.VMEM((nbuf, _CHUNK, d_train), jnp.float32),
            [pltpu.SemaphoreType.DMA] * nbuf,
            [pltpu.SemaphoreType.DMA] * nbuf,
        ],
        compiler_params=pltpu.CompilerParams(use_tc_tiling_on_sc=False),
    )
    def sc_gather(freeze_hbm, train_hbm, idx_hbm, out_hbm,
                  idx_v, rows_f, rows_t, gsems, wsems):
        wid = lax.axis_index("s") * _NUM_CORES + lax.axis_index("c")
        base = wid * bpw
        pltpu.sync_copy(idx_hbm.at[pl.ds(base, bpw)], idx_v)

        def start_gathers(c, b):
            off = c * _CHUNK
            idx_c = idx_v.at[pl.ds(off, _CHUNK)]
            pltpu.async_copy(freeze_hbm.at[idx_c], rows_f.at[b], gsems[b])
            pltpu.async_copy(train_hbm.at[idx_c], rows_t.at[b], gsems[b])

        def drain_write(b):
            pltpu.make_async_copy(
                rows_f.at[b], out_hbm.at[pl.ds(base, _CHUNK), pl.ds(0, d_freeze)],
                wsems[b]).wait()
            pltpu.make_async_copy(
                rows_t.at[b], out_hbm.at[pl.ds(base, _CHUNK), pl.ds(d_freeze, d_train)],
                wsems[b]).wait()

        def group_body(j, carry):
            c0 = j * nbuf
            for b in range(nbuf):

                @pl.when(j > 0)
                def _():
                    drain_write(b)

                start_gathers(c0 + b, b)
            for b in range(nbuf):
                pltpu.make_async_copy(
                    freeze_hbm.at[pl.ds(0, _CHUNK)], rows_f.at[b], gsems[b]).wait()
                pltpu.make_async_copy(
                    train_hbm.at[pl.ds(0, _CHUNK)], rows_t.at[b], gsems[b]).wait()
                row0 = base + (c0 + b) * _CHUNK
                pltpu.async_copy(
                    rows_f.at[b],
                    out_hbm.at[pl.ds(row0, _CHUNK), pl.ds(0, d_freeze)], wsems[b])
                pltpu.async_copy(
                    rows_t.at[b],
                    out_hbm.at[pl.ds(row0, _CHUNK), pl.ds(d_freeze, d_train)], wsems[b])
            return carry

        lax.fori_loop(0, n_chunks // nbuf, group_body, 0)
        for b in range(nbuf):
            drain_write(b)

    return sc_gather


def kernel(input_ids, freeze_buffer, training_part):
    b, s = input_ids.shape
    n_idx = b * s
    d_freeze = freeze_buffer.shape[1]
    d_train = training_part.shape[1]
    idx = jnp.reshape(input_ids, (n_idx,)).astype(jnp.int32)
    sc_gather = _make_sc_gather(n_idx, d_freeze, d_train)
    out = sc_gather(freeze_buffer, training_part, idx)
    return jnp.reshape(out, (b, s, d_freeze + d_train))


# speedup vs baseline: 2.1689x; 1.9755x over previous
import functools

import jax
import jax.numpy as jnp
from jax import lax
from jax.experimental import pallas as pl
from jax.experimental.pallas import tpu as pltpu
from jax.experimental.pallas import tpu_sc as plsc

_NUM_CORES = 2
_NUM_SUBCORES = 16
_NUM_WORKERS = _NUM_CORES * _NUM_SUBCORES
_CHUNK = 128
_VB = 512


@functools.lru_cache(maxsize=None)
def _make_tc_merge(d_freeze, d_train, vocab):
    d_out = d_freeze + d_train
    grid = (vocab + _VB - 1) // _VB

    def body(f_ref, t_ref, o_ref):
        f = jnp.transpose(f_ref[...], (1, 0))
        t = jnp.transpose(t_ref[...], (1, 0))
        o_ref[...] = jnp.concatenate([f, t], axis=1)

    return pl.pallas_call(
        body,
        grid=(grid,),
        in_specs=[
            pl.BlockSpec((d_freeze, _VB), lambda i: (0, i)),
            pl.BlockSpec((d_train, _VB), lambda i: (0, i)),
        ],
        out_specs=pl.BlockSpec((_VB, d_out), lambda i: (i, 0)),
        out_shape=jax.ShapeDtypeStruct((vocab, d_out), jnp.float32),
    )


@functools.lru_cache(maxsize=None)
def _make_sc_gather(n_idx, d_out):
    bpw = n_idx // _NUM_WORKERS
    n_chunks = bpw // _CHUNK
    nbuf = 2
    assert n_idx % _NUM_WORKERS == 0 and bpw % _CHUNK == 0
    assert n_chunks % nbuf == 0

    mesh = plsc.VectorSubcoreMesh(
        core_axis_name="c", subcore_axis_name="s",
        num_cores=_NUM_CORES, num_subcores=_NUM_SUBCORES)

    @functools.partial(
        pl.kernel,
        out_type=jax.ShapeDtypeStruct((n_idx, d_out), jnp.float32),
        mesh=mesh,
        scratch_types=[
            pltpu.VMEM((bpw,), jnp.int32),
            pltpu.VMEM((nbuf, _CHUNK, d_out), jnp.float32),
            [pltpu.SemaphoreType.DMA] * nbuf,
            [pltpu.SemaphoreType.DMA] * nbuf,
        ],
    )
    def sc_gather(table_hbm, idx_hbm, out_hbm, idx_v, rows, gsems, wsems):
        wid = lax.axis_index("s") * _NUM_CORES + lax.axis_index("c")
        base = wid * bpw
        pltpu.sync_copy(idx_hbm.at[pl.ds(base, bpw)], idx_v)

        def drain_write(b):
            pltpu.make_async_copy(
                rows.at[b], out_hbm.at[pl.ds(base, _CHUNK)], wsems[b]).wait()

        def group_body(j, carry):
            c0 = j * nbuf
            for b in range(nbuf):

                @pl.when(j > 0)
                def _():
                    drain_write(b)

                off = (c0 + b) * _CHUNK
                pltpu.async_copy(
                    table_hbm.at[idx_v.at[pl.ds(off, _CHUNK)]],
                    rows.at[b], gsems[b])
            for b in range(nbuf):
                pltpu.make_async_copy(
                    table_hbm.at[pl.ds(0, _CHUNK)], rows.at[b], gsems[b]).wait()
                row0 = base + (c0 + b) * _CHUNK
                pltpu.async_copy(rows.at[b],
                                 out_hbm.at[pl.ds(row0, _CHUNK)], wsems[b])
            return carry

        lax.fori_loop(0, n_chunks // nbuf, group_body, 0)
        for b in range(nbuf):
            drain_write(b)

    return sc_gather


def kernel(input_ids, freeze_buffer, training_part):
    b, s = input_ids.shape
    n_idx = b * s
    vocab, d_freeze = freeze_buffer.shape
    d_train = training_part.shape[1]
    d_out = d_freeze + d_train

    merged = _make_tc_merge(d_freeze, d_train, vocab)(
        jnp.transpose(freeze_buffer, (1, 0)), jnp.transpose(training_part, (1, 0)))
    idx = jnp.reshape(jnp.transpose(input_ids, (1, 0)), (n_idx,)).astype(jnp.int32)
    out = _make_sc_gather(n_idx, d_out)(merged, idx)
    return jnp.transpose(jnp.reshape(out, (s, b, d_out)), (1, 0, 2))


# speedup vs baseline: 3.4013x; 1.5682x over previous
import functools

import jax
import jax.numpy as jnp
from jax import lax
from jax.experimental import pallas as pl
from jax.experimental.pallas import tpu as pltpu
from jax.experimental.pallas import tpu_sc as plsc

_NUM_CORES = 2
_NUM_SUBCORES = 16
_NUM_WORKERS = _NUM_CORES * _NUM_SUBCORES
_CHUNK = 128
_VB = 2048


@functools.lru_cache(maxsize=None)
def _make_tc_merge(d_freeze, d_train, vocab):
    d_out = d_freeze + d_train
    grid = (vocab + _VB - 1) // _VB

    def body(f_ref, t_ref, o_ref):
        f = jnp.transpose(f_ref[...], (1, 0))
        t = jnp.transpose(t_ref[...], (1, 0))
        o_ref[...] = jnp.concatenate([f, t], axis=1)

    return pl.pallas_call(
        body,
        grid=(grid,),
        in_specs=[
            pl.BlockSpec((d_freeze, _VB), lambda i: (0, i)),
            pl.BlockSpec((d_train, _VB), lambda i: (0, i)),
        ],
        out_specs=pl.BlockSpec((_VB, d_out), lambda i: (i, 0)),
        out_shape=jax.ShapeDtypeStruct((vocab, d_out), jnp.float32),
    )


@functools.lru_cache(maxsize=None)
def _make_sc_gather(n_idx, d_out):
    bpw = n_idx // _NUM_WORKERS
    n_chunks = bpw // _CHUNK
    nbuf = 2
    assert n_idx % _NUM_WORKERS == 0 and bpw % _CHUNK == 0
    assert n_chunks % nbuf == 0

    mesh = plsc.VectorSubcoreMesh(
        core_axis_name="c", subcore_axis_name="s",
        num_cores=_NUM_CORES, num_subcores=_NUM_SUBCORES)

    @functools.partial(
        pl.kernel,
        out_type=jax.ShapeDtypeStruct((n_idx, d_out), jnp.float32),
        mesh=mesh,
        scratch_types=[
            pltpu.VMEM((bpw,), jnp.int32),
            pltpu.VMEM((nbuf, _CHUNK, d_out), jnp.float32),
            [pltpu.SemaphoreType.DMA] * nbuf,
            [pltpu.SemaphoreType.DMA] * nbuf,
        ],
    )
    def sc_gather(table_hbm, idx_hbm, out_hbm, idx_v, rows, gsems, wsems):
        wid = lax.axis_index("s") * _NUM_CORES + lax.axis_index("c")
        base = wid * bpw
        pltpu.sync_copy(idx_hbm.at[pl.ds(base, bpw)], idx_v)

        def drain_write(b):
            pltpu.make_async_copy(
                rows.at[b], out_hbm.at[pl.ds(base, _CHUNK)], wsems[b]).wait()

        def group_body(j, carry):
            c0 = j * nbuf
            for b in range(nbuf):

                @pl.when(j > 0)
                def _():
                    drain_write(b)

                off = (c0 + b) * _CHUNK
                pltpu.async_copy(
                    table_hbm.at[idx_v.at[pl.ds(off, _CHUNK)]],
                    rows.at[b], gsems[b])
            for b in range(nbuf):
                pltpu.make_async_copy(
                    table_hbm.at[pl.ds(0, _CHUNK)], rows.at[b], gsems[b]).wait()
                row0 = base + (c0 + b) * _CHUNK
                pltpu.async_copy(rows.at[b],
                                 out_hbm.at[pl.ds(row0, _CHUNK)], wsems[b])
            return carry

        lax.fori_loop(0, n_chunks // nbuf, group_body, 0)
        for b in range(nbuf):
            drain_write(b)

    return sc_gather


def kernel(input_ids, freeze_buffer, training_part):
    b, s = input_ids.shape
    n_idx = b * s
    vocab, d_freeze = freeze_buffer.shape
    d_train = training_part.shape[1]
    d_out = d_freeze + d_train

    merged = _make_tc_merge(d_freeze, d_train, vocab)(
        jnp.transpose(freeze_buffer, (1, 0)), jnp.transpose(training_part, (1, 0)))
    idx = jnp.reshape(jnp.transpose(input_ids, (1, 0)), (n_idx,)).astype(jnp.int32)
    out = _make_sc_gather(n_idx, d_out)(merged, idx)
    return jnp.transpose(jnp.reshape(out, (s, b, d_out)), (1, 0, 2))


# speedup vs baseline: 3.9275x; 1.1547x over previous
import functools

import jax
import jax.numpy as jnp
from jax import lax
from jax.experimental import pallas as pl
from jax.experimental.pallas import tpu as pltpu
from jax.experimental.pallas import tpu_sc as plsc

_NUM_CORES = 2
_NUM_SUBCORES = 16
_NUM_WORKERS = _NUM_CORES * _NUM_SUBCORES
_CHUNK = 128
_VB = 8192


@functools.lru_cache(maxsize=None)
def _make_tc_merge(d_freeze, d_train, vocab):
    d_out = d_freeze + d_train
    grid = (vocab + _VB - 1) // _VB

    def body(f_ref, t_ref, o_ref):
        f = jnp.transpose(f_ref[...], (1, 0))
        t = jnp.transpose(t_ref[...], (1, 0))
        o_ref[...] = jnp.concatenate([f, t], axis=1)

    return pl.pallas_call(
        body,
        grid=(grid,),
        in_specs=[
            pl.BlockSpec((d_freeze, _VB), lambda i: (0, i)),
            pl.BlockSpec((d_train, _VB), lambda i: (0, i)),
        ],
        out_specs=pl.BlockSpec((_VB, d_out), lambda i: (i, 0)),
        out_shape=jax.ShapeDtypeStruct((vocab, d_out), jnp.float32),
    )


@functools.lru_cache(maxsize=None)
def _make_sc_gather(n_idx, d_out):
    bpw = n_idx // _NUM_WORKERS
    n_chunks = bpw // _CHUNK
    nbuf = 2
    assert n_idx % _NUM_WORKERS == 0 and bpw % _CHUNK == 0
    assert n_chunks % nbuf == 0

    mesh = plsc.VectorSubcoreMesh(
        core_axis_name="c", subcore_axis_name="s",
        num_cores=_NUM_CORES, num_subcores=_NUM_SUBCORES)

    @functools.partial(
        pl.kernel,
        out_type=jax.ShapeDtypeStruct((n_idx, d_out), jnp.float32),
        mesh=mesh,
        scratch_types=[
            pltpu.VMEM((bpw,), jnp.int32),
            pltpu.VMEM((nbuf, _CHUNK, d_out), jnp.float32),
            [pltpu.SemaphoreType.DMA] * nbuf,
            [pltpu.SemaphoreType.DMA] * nbuf,
        ],
    )
    def sc_gather(table_hbm, idx_hbm, out_hbm, idx_v, rows, gsems, wsems):
        wid = lax.axis_index("s") * _NUM_CORES + lax.axis_index("c")
        base = wid * bpw
        pltpu.sync_copy(idx_hbm.at[pl.ds(base, bpw)], idx_v)

        def drain_write(b):
            pltpu.make_async_copy(
                rows.at[b], out_hbm.at[pl.ds(base, _CHUNK)], wsems[b]).wait()

        def group_body(j, carry):
            c0 = j * nbuf
            for b in range(nbuf):

                @pl.when(j > 0)
                def _():
                    drain_write(b)

                off = (c0 + b) * _CHUNK
                pltpu.async_copy(
                    table_hbm.at[idx_v.at[pl.ds(off, _CHUNK)]],
                    rows.at[b], gsems[b])
            for b in range(nbuf):
                pltpu.make_async_copy(
                    table_hbm.at[pl.ds(0, _CHUNK)], rows.at[b], gsems[b]).wait()
                row0 = base + (c0 + b) * _CHUNK
                pltpu.async_copy(rows.at[b],
                                 out_hbm.at[pl.ds(row0, _CHUNK)], wsems[b])
            return carry

        lax.fori_loop(0, n_chunks // nbuf, group_body, 0)
        for b in range(nbuf):
            drain_write(b)

    return sc_gather


def kernel(input_ids, freeze_buffer, training_part):
    b, s = input_ids.shape
    n_idx = b * s
    vocab, d_freeze = freeze_buffer.shape
    d_train = training_part.shape[1]
    d_out = d_freeze + d_train

    merged = _make_tc_merge(d_freeze, d_train, vocab)(
        jnp.transpose(freeze_buffer, (1, 0)), jnp.transpose(training_part, (1, 0)))
    idx = jnp.reshape(jnp.transpose(input_ids, (1, 0)), (n_idx,)).astype(jnp.int32)
    out = _make_sc_gather(n_idx, d_out)(merged, idx)
    return jnp.transpose(jnp.reshape(out, (s, b, d_out)), (1, 0, 2))
